# baseline (speedup 1.0000x reference)
import functools

import jax
import jax.numpy as jnp
from jax import lax
from jax.experimental import pallas as pl
from jax.experimental.pallas import tpu as pltpu
from jax.experimental.pallas import tpu_sc as plsc

C = 96
BINS = 1000
B = 8
H = 384
W = 384
N_PER_CH = B * H * W
NW = 32
CH_PER_W = C // NW
R = 64
CHUNKS_PER_BLOCK = H // R
NCHUNK = B * CHUNKS_PER_BLOCK
HIST_PAD = 1024
RED_W = 1008


def _sc_stats(x2):
  mesh = plsc.VectorSubcoreMesh(core_axis_name="c", subcore_axis_name="s")

  @functools.partial(
      pl.kernel,
      out_type=[
          jax.ShapeDtypeStruct((NW, 4, 16), jnp.float32),
          jax.ShapeDtypeStruct((NW, CH_PER_W, RED_W), jnp.float32),
      ],
      mesh=mesh,
      compiler_params=pltpu.CompilerParams(needs_layout_passes=False),
      scratch_types=[
          pltpu.VMEM((R, W), jnp.float32),
          pltpu.VMEM((R, W), jnp.float32),
          pltpu.VMEM((16 * HIST_PAD,), jnp.int32),
          pltpu.VMEM((1, 1, RED_W), jnp.float32),
          pltpu.VMEM((1, 4, 16), jnp.float32),
          pltpu.SemaphoreType.DMA,
          pltpu.SemaphoreType.DMA,
      ],
  )
  def k(x_hbm, stats_out, hist_out, buf0, buf1, hist, red, statbuf,
        sem0, sem1):
    wid = lax.axis_index("s") * 2 + lax.axis_index("c")
    lane = lax.broadcasted_iota(jnp.int32, (16,), 0)
    laneoff = lane * HIST_PAD
    ones = jnp.ones((16,), jnp.int32)
    zeros = jnp.zeros((16,), jnp.float32)
    inf = jnp.full((16,), jnp.inf, jnp.float32)
    bufs = (buf0, buf1)
    sems = (sem0, sem1)

    def chunk_row0(ch, kc):
      b = kc // CHUNKS_PER_BLOCK
      r = kc % CHUNKS_PER_BLOCK
      return (b * C + ch) * H + r * R

    def start(ch, kc, j):
      pltpu.async_copy(
          x_hbm.at[pl.ds(chunk_row0(ch, kc), R), :], bufs[j], sems[j])

    def wait(j):
      pltpu.make_async_copy(
          x_hbm.at[pl.ds(0, R), :], bufs[j], sems[j]).wait()

    def process(buf, carry):
      def row_body(i, c):
        r = i >> 3
        cb = (i & 7) * 48
        s, ss, mn, mx = c
        for kk in range(3):
          v = buf[r, pl.ds(cb + kk * 16, 16)]
          s = s + v
          ss = ss + v * v
          mn = jnp.minimum(mn, v)
          mx = jnp.maximum(mx, v)
          scaled = v * 100.0 + 500.0
          idx = scaled.astype(jnp.int32)
          idx = jnp.minimum(idx, BINS - 1)
          m = jnp.abs(v) <= 5.0
          plsc.addupdate_scatter(hist, [idx + laneoff], ones, mask=m)
        return (s, ss, mn, mx)
      return plsc.parallel_loop(0, R * 8, unroll=16, carry=carry)(row_body)

    mean_l = zeros
    var_l = zeros
    min_l = zeros
    max_l = zeros
    for cl in range(CH_PER_W):
      ch = wid * CH_PER_W + cl

      izeros = jnp.zeros((16,), jnp.int32)

      def zb(j, _):
        hist[pl.ds(j * 16, 16)] = izeros
        return 0
      lax.fori_loop(0, 16 * HIST_PAD // 16, zb, 0)

      carry = (zeros, zeros, inf, -inf)
      start(ch, 0, 0)

      def g_body(g, c):
        start(ch, 2 * g + 1, 1)
        wait(0)
        c = process(buf0, c)

        @pl.when(g < NCHUNK // 2 - 1)
        def _():
          start(ch, 2 * g + 2, 0)

        wait(1)
        c = process(buf1, c)
        return c

      carry = lax.fori_loop(0, NCHUNK // 2, g_body, carry)
      s, ss, mn, mx = carry

      sum_t = jnp.broadcast_to(jnp.sum(s), (16,))
      ssq_t = jnp.broadcast_to(jnp.sum(ss), (16,))
      mn_t = jnp.broadcast_to(jnp.min(mn), (16,))
      mx_t = jnp.broadcast_to(jnp.max(mx), (16,))
      nf = jnp.float32(N_PER_CH)
      mean_v = sum_t / nf
      var_v = (ssq_t - nf * mean_v * mean_v) / jnp.float32(N_PER_CH - 1)
      sel = lane == cl
      mean_l = jnp.where(sel, mean_v, mean_l)
      var_l = jnp.where(sel, var_v, var_l)
      min_l = jnp.where(sel, mn_t, min_l)
      max_l = jnp.where(sel, mx_t, max_l)

      def rg(j, _):
        acc = hist[pl.ds(j * 16, 16)]
        for l in range(1, 16):
          acc = acc + hist[pl.ds(l * HIST_PAD + j * 16, 16)]
        red[0, 0, pl.ds(j * 16, 16)] = acc.astype(jnp.float32)
        return 0
      lax.fori_loop(0, RED_W // 16, rg, 0)
      pltpu.sync_copy(red, hist_out.at[pl.ds(wid, 1), pl.ds(cl, 1)])

    statbuf[0, 0, :] = mean_l
    statbuf[0, 1, :] = var_l
    statbuf[0, 2, :] = min_l
    statbuf[0, 3, :] = max_l
    pltpu.sync_copy(statbuf, stats_out.at[pl.ds(wid, 1)])

  return k(x2)


def kernel(x):
  x2 = x.reshape(B * C * H, W)
  stats, hist = _sc_stats(x2)
  mean = stats[:, 0, :CH_PER_W].reshape(C)
  var = stats[:, 1, :CH_PER_W].reshape(C)
  mn = stats[:, 2, :CH_PER_W].reshape(C)
  mx = stats[:, 3, :CH_PER_W].reshape(C)
  std = jnp.sqrt(var + 1e-8)
  cnt = jnp.full((1,), jnp.float32(B))
  hist_f = hist.reshape(C, RED_W)[:, :BINS]
  return (x, mean, var, std, mn, mx, cnt, hist_f)


# speedup vs baseline: 1.5544x; 1.5544x over previous
import functools

import jax
import jax.numpy as jnp
from jax import lax
from jax.experimental import pallas as pl
from jax.experimental.pallas import tpu as pltpu
from jax.experimental.pallas import tpu_sc as plsc

C = 96
BINS = 1000
B = 8
H = 384
W = 384
N_PER_CH = B * H * W
NW = 32
CH_PER_W = C // NW
R = 64
CHUNKS_PER_BLOCK = H // R
NCHUNK = B * CHUNKS_PER_BLOCK
HIST_PAD = 1024
RED_W = 1008


def _sc_hist(x2):
  mesh = plsc.VectorSubcoreMesh(core_axis_name="c", subcore_axis_name="s")

  @functools.partial(
      pl.kernel,
      out_type=jax.ShapeDtypeStruct((NW, CH_PER_W, RED_W), jnp.float32),
      mesh=mesh,
      compiler_params=pltpu.CompilerParams(needs_layout_passes=False),
      scratch_types=[
          pltpu.VMEM((R, W), jnp.float32),
          pltpu.VMEM((R, W), jnp.float32),
          pltpu.VMEM((16 * HIST_PAD,), jnp.int32),
          pltpu.VMEM((1, 1, RED_W), jnp.float32),
          pltpu.SemaphoreType.DMA,
          pltpu.SemaphoreType.DMA,
      ],
  )
  def k(x_hbm, hist_out, buf0, buf1, hist, red, sem0, sem1):
    wid = lax.axis_index("s") * 2 + lax.axis_index("c")
    lane = lax.broadcasted_iota(jnp.int32, (16,), 0)
    laneoff = lane * HIST_PAD
    ones = jnp.ones((16,), jnp.int32)
    izeros = jnp.zeros((16,), jnp.int32)
    bufs = (buf0, buf1)
    sems = (sem0, sem1)

    def chunk_row0(ch, kc):
      b = kc // CHUNKS_PER_BLOCK
      r = kc % CHUNKS_PER_BLOCK
      return (b * C + ch) * H + r * R

    def start(ch, kc, j):
      pltpu.async_copy(
          x_hbm.at[pl.ds(chunk_row0(ch, kc), R), :], bufs[j], sems[j])

    def wait(j):
      pltpu.make_async_copy(
          x_hbm.at[pl.ds(0, R), :], bufs[j], sems[j]).wait()

    def process(buf):
      def row_body(i):
        r = i >> 3
        cb = (i & 7) * 48
        for kk in range(3):
          v = buf[r, pl.ds(cb + kk * 16, 16)]
          scaled = v * 100.0 + 500.0
          idx = scaled.astype(jnp.int32)
          idx = jnp.minimum(idx, BINS - 1)
          m = jnp.abs(v) <= 5.0
          plsc.addupdate_scatter(hist, [idx + laneoff], ones, mask=m)
      plsc.parallel_loop(0, R * 8, unroll=8)(row_body)

    for cl in range(CH_PER_W):
      ch = wid * CH_PER_W + cl

      def zb(j, _):
        hist[pl.ds(j * 16, 16)] = izeros
        return 0
      lax.fori_loop(0, 16 * HIST_PAD // 16, zb, 0)

      start(ch, 0, 0)

      def g_body(g, acc):
        start(ch, 2 * g + 1, 1)
        wait(0)
        process(buf0)

        @pl.when(g < NCHUNK // 2 - 1)
        def _():
          start(ch, 2 * g + 2, 0)

        wait(1)
        process(buf1)
        return acc

      lax.fori_loop(0, NCHUNK // 2, g_body, 0)

      def rg(j, _):
        acc = hist[pl.ds(j * 16, 16)]
        for l in range(1, 16):
          acc = acc + hist[pl.ds(l * HIST_PAD + j * 16, 16)]
        red[0, 0, pl.ds(j * 16, 16)] = acc.astype(jnp.float32)
        return 0
      lax.fori_loop(0, RED_W // 16, rg, 0)
      pltpu.sync_copy(red, hist_out.at[pl.ds(wid, 1), pl.ds(cl, 1)])

  return k(x2)


CG = 12
CGW = C // CG


def _tc_stats_block(x_ref, sum_ref, ssq_ref, mn_ref, mx_ref):
  cg = pl.program_id(0)
  b = pl.program_id(1)
  xb = x_ref[0]
  ps = jnp.sum(xb, axis=(1, 2))[None, :]
  pss = jnp.sum(xb * xb, axis=(1, 2))[None, :]
  pmn = jnp.min(xb, axis=(1, 2))[None, :]
  pmx = jnp.max(xb, axis=(1, 2))[None, :]
  rowmask = lax.broadcasted_iota(jnp.int32, (CG, CGW), 0) == cg

  @pl.when(b == 0)
  def _():
    sum_ref[...] = jnp.where(rowmask, ps, sum_ref[...])
    ssq_ref[...] = jnp.where(rowmask, pss, ssq_ref[...])
    mn_ref[...] = jnp.where(rowmask, pmn, mn_ref[...])
    mx_ref[...] = jnp.where(rowmask, pmx, mx_ref[...])

  @pl.when(b > 0)
  def _():
    sum_ref[...] = jnp.where(rowmask, sum_ref[...] + ps, sum_ref[...])
    ssq_ref[...] = jnp.where(rowmask, ssq_ref[...] + pss, ssq_ref[...])
    mn_ref[...] = jnp.where(rowmask, jnp.minimum(mn_ref[...], pmn),
                            mn_ref[...])
    mx_ref[...] = jnp.where(rowmask, jnp.maximum(mx_ref[...], pmx),
                            mx_ref[...])


def _tc_stats(x):
  out = jax.ShapeDtypeStruct((CG, CGW), jnp.float32)
  ospec = pl.BlockSpec((CG, CGW), lambda cg, b: (0, 0))
  return pl.pallas_call(
      _tc_stats_block,
      grid=(CG, B),
      in_specs=[pl.BlockSpec((1, CGW, H, W), lambda cg, b: (b, cg, 0, 0))],
      out_specs=[ospec, ospec, ospec, ospec],
      out_shape=[out, out, out, out],
      compiler_params=pltpu.CompilerParams(
          dimension_semantics=("arbitrary", "arbitrary")),
  )(x)


def kernel(x):
  x2 = x.reshape(B * C * H, W)
  s2, ss2, mn2, mx2 = _tc_stats(x)
  s, ss, mn, mx = (a.reshape(C) for a in (s2, ss2, mn2, mx2))
  hist = _sc_hist(x2)
  nf = jnp.float32(N_PER_CH)
  mean = s / nf
  var = (ss - nf * mean * mean) / jnp.float32(N_PER_CH - 1)
  std = jnp.sqrt(var + 1e-8)
  cnt = jnp.full((1,), jnp.float32(B))
  hist_f = hist.reshape(C, RED_W)[:, :BINS]
  return (x, mean, var, std, mn, mx, cnt, hist_f)
